# baseline (speedup 1.0000x reference)
import functools

import jax
import jax.numpy as jnp
from jax.experimental import pallas as pl
from jax.experimental.pallas import tpu as pltpu

E = 8
K = 2
D = 768
H = 3072
_NEG = -1e30


def _router_body(x_ref, wg_ref, w_ref, aux_ref):
    x = x_ref[...]
    wg = wg_ref[...]
    scores = jax.lax.dot_general(
        x, wg, (((1,), (1,)), ((), ())), preferred_element_type=jnp.float32
    )
    T = scores.shape[0]
    idx = jax.lax.broadcasted_iota(jnp.int32, scores.shape, 1)
    m0 = jnp.max(scores, axis=1, keepdims=True)
    i0 = jnp.min(jnp.where(scores >= m0, idx, E), axis=1, keepdims=True)
    oh0 = idx == i0
    s2 = jnp.where(oh0, _NEG, scores)
    m1 = jnp.max(s2, axis=1, keepdims=True)
    i1 = jnp.min(jnp.where(s2 >= m1, idx, E), axis=1, keepdims=True)
    oh1 = idx == i1
    p0 = 1.0 / (1.0 + jnp.exp(m1 - m0))
    p1 = 1.0 - p0
    w = jnp.where(oh0, p0, 0.0) + jnp.where(oh1, p1, 0.0)
    w_ref[...] = w
    g = jnp.exp(scores - m0)
    g = g / jnp.sum(g, axis=1, keepdims=True)
    avg_g = jnp.sum(g, axis=0) * (1.0 / T)
    counts = jnp.sum(jnp.where(oh0 | oh1, 1.0, 0.0), axis=0)
    aux_ref[...] = jnp.reshape(E * jnp.sum(counts * (1.0 / T) * avg_g), (1, 1))


def _moe_body(x_ref, w_ref, w1_ref, b1_ref, w2_ref, b2_ref, w3_ref, b3_ref,
              o_ref):
    e = pl.program_id(1)
    h = pl.program_id(2)

    @pl.when((e == 0) & (h == 0))
    def _init():
        o_ref[...] = jnp.zeros_like(o_ref)

    x = x_ref[...]
    a = jax.lax.dot_general(
        x, w1_ref[0], (((1,), (1,)), ((), ())), preferred_element_type=jnp.float32
    ) + b1_ref[0]
    b = jax.lax.dot_general(
        x, w2_ref[0], (((1,), (1,)), ((), ())), preferred_element_type=jnp.float32
    ) + b2_ref[0]
    hact = (a * jax.nn.sigmoid(a) * b).astype(jnp.bfloat16)
    y = jax.lax.dot_general(
        hact, w3_ref[0], (((1,), (1,)), ((), ())), preferred_element_type=jnp.float32
    )
    lane = jax.lax.broadcasted_iota(jnp.int32, w_ref.shape, 1)
    we = jnp.sum(jnp.where(lane == e, w_ref[...], 0.0), axis=1, keepdims=True)

    @pl.when(h == 0)
    def _bias3():
        o_ref[...] += we * b3_ref[0]

    o_ref[...] += y * we


def kernel(x, Wg, W1, b1, W2, b2, W3, b3):
    B, S, _ = x.shape
    T = B * S
    x_flat = x.reshape(T, D)

    w_mat, aux = pl.pallas_call(
        _router_body,
        out_shape=(
            jax.ShapeDtypeStruct((T, E), jnp.float32),
            jax.ShapeDtypeStruct((1, 1), jnp.float32),
        ),
        in_specs=[
            pl.BlockSpec((T, D), lambda: (0, 0)),
            pl.BlockSpec((E, D), lambda: (0, 0)),
        ],
        out_specs=(
            pl.BlockSpec((T, E), lambda: (0, 0)),
            pl.BlockSpec((1, 1), lambda: (0, 0)),
        ),
    )(x_flat, Wg)

    BT = 512
    BH = 768
    grid = (T // BT, E, H // BH)
    out = pl.pallas_call(
        _moe_body,
        grid=grid,
        in_specs=[
            pl.BlockSpec((BT, D), lambda t, e, h: (t, 0)),
            pl.BlockSpec((BT, E), lambda t, e, h: (t, 0)),
            pl.BlockSpec((1, BH, D), lambda t, e, h: (e, h, 0)),
            pl.BlockSpec((1, 1, BH), lambda t, e, h: (e, 0, h)),
            pl.BlockSpec((1, BH, D), lambda t, e, h: (e, h, 0)),
            pl.BlockSpec((1, 1, BH), lambda t, e, h: (e, 0, h)),
            pl.BlockSpec((1, D, BH), lambda t, e, h: (e, 0, h)),
            pl.BlockSpec((1, 1, D), lambda t, e, h: (e, 0, 0)),
        ],
        out_specs=pl.BlockSpec((BT, D), lambda t, e, h: (t, 0)),
        out_shape=jax.ShapeDtypeStruct((T, D), jnp.float32),
        compiler_params=pltpu.CompilerParams(
            dimension_semantics=("parallel", "arbitrary", "arbitrary"),
        ),
    )(x_flat.astype(jnp.bfloat16), w_mat, W1.astype(jnp.bfloat16),
      b1.reshape(E, 1, H), W2.astype(jnp.bfloat16), b2.reshape(E, 1, H),
      W3.astype(jnp.bfloat16), b3.reshape(E, 1, D))

    return out.reshape(B, S, D), aux[0, 0]


# speedup vs baseline: 1.4236x; 1.4236x over previous
import functools

import jax
import jax.numpy as jnp
from jax.experimental import pallas as pl
from jax.experimental.pallas import tpu as pltpu

E = 8
K = 2
D = 768
H = 3072
_NEG = -1e30


def _router_body(x_ref, wg_ref, w_ref, aux_ref):
    x = x_ref[...]
    wg = wg_ref[...]
    scores = jax.lax.dot_general(
        x, wg, (((1,), (1,)), ((), ())), preferred_element_type=jnp.float32
    )
    T = scores.shape[0]
    idx = jax.lax.broadcasted_iota(jnp.int32, scores.shape, 1)
    m0 = jnp.max(scores, axis=1, keepdims=True)
    i0 = jnp.min(jnp.where(scores >= m0, idx, E), axis=1, keepdims=True)
    oh0 = idx == i0
    s2 = jnp.where(oh0, _NEG, scores)
    m1 = jnp.max(s2, axis=1, keepdims=True)
    i1 = jnp.min(jnp.where(s2 >= m1, idx, E), axis=1, keepdims=True)
    oh1 = idx == i1
    p0 = 1.0 / (1.0 + jnp.exp(m1 - m0))
    p1 = 1.0 - p0
    w = jnp.where(oh0, p0, 0.0) + jnp.where(oh1, p1, 0.0)
    w_ref[...] = w
    g = jnp.exp(scores - m0)
    g = g / jnp.sum(g, axis=1, keepdims=True)
    avg_g = jnp.sum(g, axis=0) * (1.0 / T)
    counts = jnp.sum(jnp.where(oh0 | oh1, 1.0, 0.0), axis=0)
    aux_ref[...] = jnp.reshape(E * jnp.sum(counts * (1.0 / T) * avg_g), (1, 1))


def _moe_body(x_ref, w_ref, w1_ref, b1_ref, w2_ref, b2_ref, w3_ref, b3_ref,
              o_ref):
    e = pl.program_id(0)
    h = pl.program_id(1)

    @pl.when((e == 0) & (h == 0))
    def _init():
        o_ref[...] = jnp.zeros_like(o_ref)

    x = x_ref[...]
    w1 = w1_ref[0].astype(jnp.bfloat16)
    w2 = w2_ref[0].astype(jnp.bfloat16)
    w3 = w3_ref[0].astype(jnp.bfloat16)
    a = jax.lax.dot_general(
        x, w1, (((1,), (1,)), ((), ())), preferred_element_type=jnp.float32
    ) + b1_ref[0]
    b = jax.lax.dot_general(
        x, w2, (((1,), (1,)), ((), ())), preferred_element_type=jnp.float32
    ) + b2_ref[0]
    hact = (a * jax.nn.sigmoid(a) * b).astype(jnp.bfloat16)
    y = jax.lax.dot_general(
        hact, w3, (((1,), (1,)), ((), ())), preferred_element_type=jnp.float32
    )
    lane = jax.lax.broadcasted_iota(jnp.int32, w_ref.shape, 1)
    we = jnp.sum(jnp.where(lane == e, w_ref[...], 0.0), axis=1, keepdims=True)

    @pl.when(h == 0)
    def _bias3():
        o_ref[...] += we * b3_ref[0]

    o_ref[...] += y * we


def kernel(x, Wg, W1, b1, W2, b2, W3, b3):
    B, S, _ = x.shape
    T = B * S
    x_flat = x.reshape(T, D)

    w_mat, aux = pl.pallas_call(
        _router_body,
        out_shape=(
            jax.ShapeDtypeStruct((T, E), jnp.float32),
            jax.ShapeDtypeStruct((1, 1), jnp.float32),
        ),
        in_specs=[
            pl.BlockSpec((T, D), lambda: (0, 0)),
            pl.BlockSpec((E, D), lambda: (0, 0)),
        ],
        out_specs=(
            pl.BlockSpec((T, E), lambda: (0, 0)),
            pl.BlockSpec((1, 1), lambda: (0, 0)),
        ),
    )(x_flat, Wg)

    BH = 512
    grid = (E, H // BH)
    out = pl.pallas_call(
        _moe_body,
        grid=grid,
        in_specs=[
            pl.BlockSpec((T, D), lambda e, h: (0, 0)),
            pl.BlockSpec((T, E), lambda e, h: (0, 0)),
            pl.BlockSpec((1, BH, D), lambda e, h: (e, h, 0)),
            pl.BlockSpec((1, 1, BH), lambda e, h: (e, 0, h)),
            pl.BlockSpec((1, BH, D), lambda e, h: (e, h, 0)),
            pl.BlockSpec((1, 1, BH), lambda e, h: (e, 0, h)),
            pl.BlockSpec((1, D, BH), lambda e, h: (e, 0, h)),
            pl.BlockSpec((1, 1, D), lambda e, h: (e, 0, 0)),
        ],
        out_specs=pl.BlockSpec((T, D), lambda e, h: (0, 0)),
        out_shape=jax.ShapeDtypeStruct((T, D), jnp.float32),
        compiler_params=pltpu.CompilerParams(
            dimension_semantics=("arbitrary", "arbitrary"),
        ),
    )(x_flat.astype(jnp.bfloat16), w_mat, W1, b1.reshape(E, 1, H), W2,
      b2.reshape(E, 1, H), W3, b3.reshape(E, 1, D))

    return out.reshape(B, S, D), aux[0, 0]
